# Initial kernel scaffold; baseline (speedup 1.0000x reference)
import functools

import jax
import jax.numpy as jnp
from jax import lax
from jax.experimental import pallas as pl
from jax.experimental.pallas import tpu as pltpu
from jax.experimental.pallas import tpu_sc as plsc

N = 10000
E = 320000
D = 128
ED = 16
CATS = 512
G = 32

NC = 2
NS = 16
CH = 128
CPW = 80
EPC = E // NC
EPC_PAD = NS * CPW * CH
NPAD = 10240
RPW = NPAD // NS
OPW = N // NS
HB = 80
HCH = 4
BLK = 1000
NB = N // BLK

_mesh = plsc.VectorSubcoreMesh(core_axis_name="c", subcore_axis_name="s")


def _mm(a, b):
    return lax.dot_general(
        a, b, (((1,), (0,)), ((), ())),
        precision=lax.Precision.HIGHEST,
        preferred_element_type=jnp.float32)


@functools.partial(
    pl.kernel,
    out_type=[
        jax.ShapeDtypeStruct((NC, N, 2 * ED), jnp.float32),
        jax.ShapeDtypeStruct((NPAD, D), jnp.float32),
    ],
    mesh=_mesh,
    scratch_types=[
        pltpu.VMEM((CH, 2 * ED), jnp.float32),
        pltpu.VMEM((CH,), jnp.int32),
        pltpu.VMEM((HB,), jnp.int32),
        pltpu.VMEM((HB, D), jnp.float32),
        pltpu.VMEM_SHARED((NPAD, 2 * ED), jnp.float32),
        pltpu.SemaphoreType.DMA,
    ])
def _sc_pre(eatt_hbm, dst_hbm, t_hbm, x1_hbm, eagg_out, hcat_out,
            ebuf, dbuf, xbuf, rbuf, accum, sem):
    c = lax.axis_index("c")
    s = lax.axis_index("s")
    wid = c * NS + s

    @pl.loop(0, CH)
    def _(r):
        @pl.loop(0, 2)
        def _(j):
            ebuf[r, pl.ds(j * 16, 16)] = jnp.zeros((16,), jnp.float32)

    @pl.loop(0, RPW // CH)
    def _(k):
        pltpu.sync_copy(ebuf, accum.at[pl.ds(s * RPW + k * CH, CH)])

    plsc.subcore_barrier()

    @pl.loop(0, CPW)
    def _(k):
        base = (s * CPW + k) * CH
        pltpu.sync_copy(dst_hbm.at[c].at[pl.ds(base, CH)], dbuf)
        pltpu.sync_copy(eatt_hbm.at[c].at[pl.ds(base, CH)], ebuf)
        pltpu.sync_copy(ebuf, accum.at[dbuf], add=True)

    plsc.subcore_barrier()
    pltpu.sync_copy(accum.at[pl.ds(s * OPW, OPW)],
                    eagg_out.at[c].at[pl.ds(s * OPW, OPW)])

    @pl.loop(0, HCH)
    def _(k):
        b = wid * (HCH * HB) + k * HB
        pltpu.sync_copy(x1_hbm.at[pl.ds(b, HB)], xbuf)
        pltpu.async_copy(t_hbm.at[xbuf], rbuf, sem).wait()
        pltpu.sync_copy(rbuf, hcat_out.at[pl.ds(b, HB)])


@functools.partial(
    pl.kernel,
    out_type=jax.ShapeDtypeStruct((NC, N, D), jnp.float32),
    mesh=_mesh,
    scratch_types=[
        pltpu.VMEM((CH, D), jnp.float32),
        pltpu.VMEM((CH,), jnp.int32),
        pltpu.VMEM((CH,), jnp.int32),
        pltpu.VMEM_SHARED((NPAD, D), jnp.float32),
        pltpu.SemaphoreType.DMA,
    ])
def _sc_spmv(p_hbm, src_hbm, dst_hbm, out_hbm, rbuf, sbuf, dbuf, accum, sem):
    c = lax.axis_index("c")
    s = lax.axis_index("s")

    @pl.loop(0, CH)
    def _(r):
        @pl.loop(0, D // 16)
        def _(j):
            rbuf[r, pl.ds(j * 16, 16)] = jnp.zeros((16,), jnp.float32)

    @pl.loop(0, RPW // CH)
    def _(k):
        pltpu.sync_copy(rbuf, accum.at[pl.ds(s * RPW + k * CH, CH)])

    plsc.subcore_barrier()

    @pl.loop(0, CPW)
    def _(k):
        base = (s * CPW + k) * CH
        pltpu.sync_copy(src_hbm.at[c].at[pl.ds(base, CH)], sbuf)
        pltpu.sync_copy(dst_hbm.at[c].at[pl.ds(base, CH)], dbuf)
        pltpu.async_copy(p_hbm.at[sbuf], rbuf, sem).wait()
        pltpu.sync_copy(rbuf, accum.at[dbuf], add=True)

    plsc.subcore_barrier()
    pltpu.sync_copy(accum.at[pl.ds(s * OPW, OPW)],
                    out_hbm.at[c].at[pl.ds(s * OPW, OPW)])


def _tmat_body(wcat_ref, wn2t_ref, out_ref):
    out_ref[...] = _mm(jnp.maximum(wcat_ref[...], 0.0), wn2t_ref[...])


def _tmat(wcat, wn2t):
    return pl.pallas_call(
        _tmat_body,
        out_shape=jax.ShapeDtypeStruct((CATS, D), jnp.float32))(wcat, wn2t)


_EB = 4096


def _eatt_body(ef_ref, wet_ref, be_ref, out_ref):
    x = jnp.maximum(_mm(ef_ref[...], wet_ref[...]) + be_ref[...], 0.0)
    ones_col = (lax.broadcasted_iota(jnp.int32, (_EB, ED), 1) == 0)
    out_ref[...] = jnp.concatenate([x, ones_col.astype(jnp.float32)], axis=1)


def _eatt(ef_pad, wet, be):
    ne = ef_pad.shape[0]
    return pl.pallas_call(
        _eatt_body,
        grid=(ne // _EB,),
        in_specs=[
            pl.BlockSpec((_EB, ED), lambda i: (i, 0)),
            pl.BlockSpec((ED, ED), lambda i: (0, 0)),
            pl.BlockSpec((1, ED), lambda i: (0, 0)),
        ],
        out_specs=pl.BlockSpec((_EB, 2 * ED), lambda i: (i, 0)),
        out_shape=jax.ShapeDtypeStruct((ne, 2 * ED), jnp.float32))(
            ef_pad, wet, be)


def _enc_body(x2_ref, hcat_ref, e0_ref, e1_ref, wgt_ref, bg_ref, wn1t_ref,
              bn_ref, wjt_ref, node_ref, proj_ref, ed_ref):
    hg = jnp.maximum(_mm(x2_ref[...], wgt_ref[...]) + bg_ref[...], 0.0)
    nd = jnp.maximum(_mm(hg, wn1t_ref[...]) + hcat_ref[...] + bn_ref[...], 0.0)
    node_ref[...] = nd
    proj_ref[...] = _mm(nd, wjt_ref[...])
    ed_ref[...] = e0_ref[...] + e1_ref[...]


def _enc(x2, hcat, e0, e1, wgt, bg, wn1t, bn, wjt):
    return pl.pallas_call(
        _enc_body,
        grid=(NB,),
        in_specs=[
            pl.BlockSpec((BLK, 32), lambda i: (i, 0)),
            pl.BlockSpec((BLK, D), lambda i: (i, 0)),
            pl.BlockSpec((BLK, 2 * ED), lambda i: (i, 0)),
            pl.BlockSpec((BLK, 2 * ED), lambda i: (i, 0)),
            pl.BlockSpec((32, D), lambda i: (0, 0)),
            pl.BlockSpec((1, D), lambda i: (0, 0)),
            pl.BlockSpec((D, D), lambda i: (0, 0)),
            pl.BlockSpec((1, D), lambda i: (0, 0)),
            pl.BlockSpec((D, D), lambda i: (0, 0)),
        ],
        out_specs=[
            pl.BlockSpec((BLK, D), lambda i: (i, 0)),
            pl.BlockSpec((BLK, D), lambda i: (i, 0)),
            pl.BlockSpec((BLK, 2 * ED), lambda i: (i, 0)),
        ],
        out_shape=[
            jax.ShapeDtypeStruct((N, D), jnp.float32),
            jax.ShapeDtypeStruct((N, D), jnp.float32),
            jax.ShapeDtypeStruct((N, 2 * ED), jnp.float32),
        ])(x2, hcat, e0, e1, wgt, bg, wn1t, bn, wjt)


def _post_body(node_ref, s0_ref, s1_ref, ed_ref, wit_ref, bm_ref, we2t_ref,
               wiht_ref, bih_ref, whht_ref, bhh_ref, h_ref, stats_ref):
    nd = node_ref[...]
    deg = ed_ref[:, ED:ED + 1]
    aggr = (deg * (_mm(nd, wit_ref[...]) + bm_ref[...])
            + s0_ref[...] + s1_ref[...]
            + _mm(ed_ref[:, 0:ED], we2t_ref[...]))
    gi = _mm(aggr, wiht_ref[...]) + bih_ref[...]
    gh = _mm(nd, whht_ref[...]) + bhh_ref[...]
    r = jax.nn.sigmoid(gi[:, 0:D] + gh[:, 0:D])
    z = jax.nn.sigmoid(gi[:, D:2 * D] + gh[:, D:2 * D])
    n = jnp.tanh(gi[:, 2 * D:] + r * gh[:, 2 * D:])
    h = (1.0 - z) * n + z * nd
    h_ref[...] = h
    hs = jnp.sum(h, axis=0)
    h2s = jnp.sum(h * h, axis=0)
    upd = jnp.concatenate(
        [hs[None, :], h2s[None, :], jnp.zeros((6, D), jnp.float32)], axis=0)

    @pl.when(pl.program_id(0) == 0)
    def _():
        stats_ref[...] = jnp.zeros((8, D), jnp.float32)

    stats_ref[...] += upd


def _post(node, s0, s1, ed, wit, bm, we2t, wiht, bih, whht, bhh):
    return pl.pallas_call(
        _post_body,
        grid=(NB,),
        in_specs=[
            pl.BlockSpec((BLK, D), lambda i: (i, 0)),
            pl.BlockSpec((BLK, D), lambda i: (i, 0)),
            pl.BlockSpec((BLK, D), lambda i: (i, 0)),
            pl.BlockSpec((BLK, 2 * ED), lambda i: (i, 0)),
            pl.BlockSpec((D, D), lambda i: (0, 0)),
            pl.BlockSpec((1, D), lambda i: (0, 0)),
            pl.BlockSpec((ED, D), lambda i: (0, 0)),
            pl.BlockSpec((D, 3 * D), lambda i: (0, 0)),
            pl.BlockSpec((1, 3 * D), lambda i: (0, 0)),
            pl.BlockSpec((D, 3 * D), lambda i: (0, 0)),
            pl.BlockSpec((1, 3 * D), lambda i: (0, 0)),
        ],
        out_specs=[
            pl.BlockSpec((BLK, D), lambda i: (i, 0)),
            pl.BlockSpec((8, D), lambda i: (0, 0)),
        ],
        out_shape=[
            jax.ShapeDtypeStruct((N, D), jnp.float32),
            jax.ShapeDtypeStruct((8, D), jnp.float32),
        ])(node, s0, s1, ed, wit, bm, we2t, wiht, bih, whht, bhh)


def _norm_common(h_ref, stats_ref, gamma_ref, beta_ref):
    mean = stats_ref[0:1, :] * (1.0 / N)
    ex2 = stats_ref[1:2, :] * (1.0 / N)
    var = ex2 - mean * mean
    return ((h_ref[...] - mean) * lax.rsqrt(var + 1e-5)
            * gamma_ref[...] + beta_ref[...])


def _norm_proj_body(h_ref, stats_ref, gamma_ref, beta_ref, wjt_ref,
                    node_ref, proj_ref):
    nd = _norm_common(h_ref, stats_ref, gamma_ref, beta_ref)
    node_ref[...] = nd
    proj_ref[...] = _mm(nd, wjt_ref[...])


def _norm_proj(h, stats, gamma, beta, wjt):
    return pl.pallas_call(
        _norm_proj_body,
        grid=(NB,),
        in_specs=[
            pl.BlockSpec((BLK, D), lambda i: (i, 0)),
            pl.BlockSpec((8, D), lambda i: (0, 0)),
            pl.BlockSpec((1, D), lambda i: (0, 0)),
            pl.BlockSpec((1, D), lambda i: (0, 0)),
            pl.BlockSpec((D, D), lambda i: (0, 0)),
        ],
        out_specs=[
            pl.BlockSpec((BLK, D), lambda i: (i, 0)),
            pl.BlockSpec((BLK, D), lambda i: (i, 0)),
        ],
        out_shape=[
            jax.ShapeDtypeStruct((N, D), jnp.float32),
            jax.ShapeDtypeStruct((N, D), jnp.float32),
        ])(h, stats, gamma, beta, wjt)


def _norm_last_body(h_ref, stats_ref, gamma_ref, beta_ref, node_ref):
    node_ref[...] = _norm_common(h_ref, stats_ref, gamma_ref, beta_ref)


def _norm_last(h, stats, gamma, beta):
    return pl.pallas_call(
        _norm_last_body,
        grid=(NB,),
        in_specs=[
            pl.BlockSpec((BLK, D), lambda i: (i, 0)),
            pl.BlockSpec((8, D), lambda i: (0, 0)),
            pl.BlockSpec((1, D), lambda i: (0, 0)),
            pl.BlockSpec((1, D), lambda i: (0, 0)),
        ],
        out_specs=pl.BlockSpec((BLK, D), lambda i: (i, 0)),
        out_shape=jax.ShapeDtypeStruct((N, D), jnp.float32))(
            h, stats, gamma, beta)


def _agg_body(node_ref, batch_ref, wlt_ref, bl_ref, wg2t_ref, bg2_ref,
              acc_ref):
    nd = node_ref[...]
    st = _mm(nd, wlt_ref[...]) + bl_ref[...]
    gz = _mm(nd, wg2t_ref[...]) + bg2_ref[...]
    m = jnp.max(gz, axis=1, keepdims=True)
    e = jnp.exp(gz - m)
    prob = e / jnp.sum(e, axis=1, keepdims=True)
    s = st * prob
    bt = batch_ref[0, 0, :]
    oh = (lax.broadcasted_iota(jnp.int32, (G, BLK), 0)
          == bt[None, :]).astype(jnp.float32)
    ones_col = (lax.broadcasted_iota(jnp.int32, (BLK, D), 1)
                == 0).astype(jnp.float32)
    sext = jnp.concatenate([s, ones_col], axis=1)
    upd = _mm(oh, sext)

    @pl.when(pl.program_id(0) == 0)
    def _():
        acc_ref[...] = jnp.zeros((G, 2 * D), jnp.float32)

    acc_ref[...] += upd


def _agg(node, batch3, wlt, bl, wg2t, bg2):
    return pl.pallas_call(
        _agg_body,
        grid=(NB,),
        in_specs=[
            pl.BlockSpec((BLK, D), lambda i: (i, 0)),
            pl.BlockSpec((1, 1, BLK), lambda i: (i, 0, 0)),
            pl.BlockSpec((D, D), lambda i: (0, 0)),
            pl.BlockSpec((1, D), lambda i: (0, 0)),
            pl.BlockSpec((D, D), lambda i: (0, 0)),
            pl.BlockSpec((1, D), lambda i: (0, 0)),
        ],
        out_specs=pl.BlockSpec((G, 2 * D), lambda i: (0, 0)),
        out_shape=jax.ShapeDtypeStruct((G, 2 * D), jnp.float32))(
            node, batch3, wlt, bl, wg2t, bg2)


def _fin_body(acc_ref, wft_ref, bf_ref, out_ref):
    summed = acc_ref[:, 0:D]
    cnt = jnp.maximum(acc_ref[:, D:D + 1], 1.0)
    out_ref[...] = _mm(summed / cnt, wft_ref[...]) + bf_ref[...]


def _fin(acc, wft, bf):
    return pl.pallas_call(
        _fin_body,
        out_shape=jax.ShapeDtypeStruct((G, D), jnp.float32))(acc, wft, bf)


def kernel(edge_index, x1, x2, edge_feats, batch, params):
    p = params
    src = edge_index[0]
    dst = edge_index[1]

    def padcore(a, padval):
        halves = []
        for ci in range(NC):
            h = a[ci * EPC:(ci + 1) * EPC]
            pad = jnp.full((EPC_PAD - EPC,) + a.shape[1:], padval, a.dtype)
            halves.append(jnp.concatenate([h, pad], axis=0))
        return jnp.stack(halves)

    srcp = padcore(src, 0)
    dstp = padcore(dst, N)
    efp = padcore(edge_feats, 0.0)
    x1p = jnp.concatenate(
        [x1[:, 0], jnp.zeros((NPAD - N,), jnp.int32)])
    batch3 = batch.reshape(NB, 1, BLK)

    wn = p["node"]["w"]
    wn1t = wn[:, 0:D].T
    wn2t = wn[:, D:2 * D].T
    wgt = p["geom"]["w"].T
    bg = p["geom"]["b"].reshape(1, D)
    bn = p["node"]["b"].reshape(1, D)
    wet = p["edge"]["w"].T
    be = p["edge"]["b"].reshape(1, ED)

    lw = []
    for lp in p["layers"]:
        wm = lp["msg"]["w"]
        lw.append(dict(
            wit=wm[:, 0:D].T,
            wjt=wm[:, D:2 * D].T,
            we2t=wm[:, 2 * D:].T,
            bm=lp["msg"]["b"].reshape(1, D),
            wiht=lp["W_ih"].T,
            bih=lp["b_ih"].reshape(1, 3 * D),
            whht=lp["W_hh"].T,
            bhh=lp["b_hh"].reshape(1, 3 * D),
            gamma=lp["gamma"].reshape(1, D),
            beta=lp["beta"].reshape(1, D),
        ))

    t_table = _tmat(p["W_cat"], wn2t)
    eatt_aug = _eatt(efp.reshape(NC * EPC_PAD, ED), wet, be)
    eagg_p, hcat = _sc_pre(
        eatt_aug.reshape(NC, EPC_PAD, 2 * ED), dstp, t_table, x1p)

    node, proj, ed = _enc(
        x2, hcat[:N], eagg_p[0], eagg_p[1],
        wgt, bg, wn1t, bn, lw[0]["wjt"])

    for li, w in enumerate(lw):
        sp = _sc_spmv(proj, srcp, dstp)
        h, stats = _post(node, sp[0], sp[1], ed, w["wit"], w["bm"],
                         w["we2t"], w["wiht"], w["bih"], w["whht"], w["bhh"])
        if li + 1 < len(lw):
            node, proj = _norm_proj(h, stats, w["gamma"], w["beta"],
                                    lw[li + 1]["wjt"])
        else:
            node = _norm_last(h, stats, w["gamma"], w["beta"])

    acc = _agg(node, batch3,
               p["agg_lin"]["w"].T, p["agg_lin"]["b"].reshape(1, D),
               p["agg_gate"]["w"].T, p["agg_gate"]["b"].reshape(1, D))
    graph = _fin(acc, p["agg_final"]["w"].T, p["agg_final"]["b"].reshape(1, D))
    return (node, graph)


# speedup vs baseline: 3.1309x; 3.1309x over previous
import functools

import jax
import jax.numpy as jnp
from jax import lax
from jax.experimental import pallas as pl
from jax.experimental.pallas import tpu as pltpu
from jax.experimental.pallas import tpu_sc as plsc

N = 10000
E = 320000
D = 128
ED = 16
CATS = 512
G = 32

NC = 2
NS = 16
CH = 128
CPW = 80
EPC = E // NC
EPC_PAD = NS * CPW * CH
NPAD = 10240
RPW = NPAD // NS
OPW = NPAD // NS
HB = 80
HCH = 4
BLK = 1000
NB = N // BLK

def _mm(a, b):
    return lax.dot_general(
        a, b, (((1,), (0,)), ((), ())),
        preferred_element_type=jnp.float32)


@functools.cache
def _sc_kernels():
    mesh = plsc.VectorSubcoreMesh(core_axis_name="c", subcore_axis_name="s",
                                  num_cores=NC, num_subcores=NS)

    @functools.partial(
        pl.kernel,
        out_type=[
            jax.ShapeDtypeStruct((NC, NPAD, D), jnp.float32),
            jax.ShapeDtypeStruct((NPAD, D), jnp.float32),
        ],
        mesh=mesh,
        scratch_types=[
            pltpu.VMEM((CH, D), jnp.float32),
            pltpu.VMEM((CH,), jnp.int32),
            pltpu.VMEM((HB,), jnp.int32),
            pltpu.VMEM((HB, D), jnp.float32),
            pltpu.VMEM_SHARED((NPAD, D), jnp.float32),
            pltpu.SemaphoreType.DMA,
        ])
    def sc_pre(eatt_hbm, dst_hbm, t_hbm, x1_hbm, eagg_out, hcat_out,
               ebuf, dbuf, xbuf, rbuf, accum, sem):
        c = lax.axis_index("c")
        s = lax.axis_index("s")
        wid = c * NS + s

        @pl.loop(0, CH)
        def _(r):
            @pl.loop(0, D // 16)
            def _(j):
                ebuf[r, pl.ds(j * 16, 16)] = jnp.zeros((16,), jnp.float32)

        @pl.loop(0, RPW // CH)
        def _(k):
            pltpu.sync_copy(ebuf, accum.at[pl.ds(s * RPW + k * CH, CH)])

        plsc.subcore_barrier()

        @pl.loop(0, CPW)
        def _(k):
            base = (s * CPW + k) * CH
            pltpu.sync_copy(dst_hbm.at[c].at[pl.ds(base, CH)], dbuf)
            pltpu.sync_copy(eatt_hbm.at[c].at[pl.ds(base, CH)], ebuf)
            pltpu.sync_copy(ebuf, accum.at[dbuf], add=True)

        plsc.subcore_barrier()
        pltpu.sync_copy(accum.at[pl.ds(s * OPW, OPW)],
                        eagg_out.at[c].at[pl.ds(s * OPW, OPW)])

        @pl.loop(0, HCH)
        def _(k):
            b = wid * (HCH * HB) + k * HB
            pltpu.sync_copy(x1_hbm.at[pl.ds(b, HB)], xbuf)
            pltpu.async_copy(t_hbm.at[xbuf], rbuf, sem).wait()
            pltpu.sync_copy(rbuf, hcat_out.at[pl.ds(b, HB)])

    @functools.partial(
        pl.kernel,
        out_type=jax.ShapeDtypeStruct((NC, NPAD, D), jnp.float32),
        mesh=mesh,
        scratch_types=[
            pltpu.VMEM((CH, D), jnp.float32),
            pltpu.VMEM((CH,), jnp.int32),
            pltpu.VMEM((CH,), jnp.int32),
            pltpu.VMEM_SHARED((NPAD, D), jnp.float32),
            pltpu.SemaphoreType.DMA,
        ])
    def sc_spmv(p_hbm, src_hbm, dst_hbm, out_hbm, rbuf, sbuf, dbuf,
                accum, sem):
        c = lax.axis_index("c")
        s = lax.axis_index("s")

        @pl.loop(0, CH)
        def _(r):
            @pl.loop(0, D // 16)
            def _(j):
                rbuf[r, pl.ds(j * 16, 16)] = jnp.zeros((16,), jnp.float32)

        @pl.loop(0, RPW // CH)
        def _(k):
            pltpu.sync_copy(rbuf, accum.at[pl.ds(s * RPW + k * CH, CH)])

        plsc.subcore_barrier()

        @pl.loop(0, CPW)
        def _(k):
            base = (s * CPW + k) * CH
            pltpu.sync_copy(src_hbm.at[c].at[pl.ds(base, CH)], sbuf)
            pltpu.sync_copy(dst_hbm.at[c].at[pl.ds(base, CH)], dbuf)
            pltpu.async_copy(p_hbm.at[sbuf], rbuf, sem).wait()
            pltpu.sync_copy(rbuf, accum.at[dbuf], add=True)

        plsc.subcore_barrier()
        pltpu.sync_copy(accum.at[pl.ds(s * OPW, OPW)],
                        out_hbm.at[c].at[pl.ds(s * OPW, OPW)])

    return sc_pre, sc_spmv


def _sc_pre(eatt3, dstp, t_table, x1p):
    return _sc_kernels()[0](eatt3, dstp, t_table, x1p)


def _sc_spmv(proj, srcp, dstp):
    return _sc_kernels()[1](proj, srcp, dstp)


def _tmat_body(wcat_ref, out_ref):
    out_ref[...] = jnp.maximum(wcat_ref[...], 0.0)


def _tmat(wcat):
    return pl.pallas_call(
        _tmat_body,
        out_shape=jax.ShapeDtypeStruct((CATS, D), jnp.float32))(wcat)


_EB = 4096


def _eatt_body(ef_ref, wet_ref, be_ref, out_ref):
    x = jnp.maximum(_mm(ef_ref[...], wet_ref[...]) + be_ref[...], 0.0)
    pad = (lax.broadcasted_iota(jnp.int32, (_EB, D - ED), 1) == 0)
    out_ref[...] = jnp.concatenate([x, pad.astype(jnp.float32)], axis=1)


def _eatt(ef_pad, wet, be):
    ne = ef_pad.shape[0]
    return pl.pallas_call(
        _eatt_body,
        grid=(ne // _EB,),
        in_specs=[
            pl.BlockSpec((_EB, ED), lambda i: (i, 0)),
            pl.BlockSpec((ED, ED), lambda i: (0, 0)),
            pl.BlockSpec((1, ED), lambda i: (0, 0)),
        ],
        out_specs=pl.BlockSpec((_EB, D), lambda i: (i, 0)),
        out_shape=jax.ShapeDtypeStruct((ne, D), jnp.float32))(
            ef_pad, wet, be)


def _enc_body(x2_ref, hcat_ref, e0_ref, e1_ref, wgt_ref, bg_ref, wnt_ref,
              bn_ref, wjt_ref, node_ref, proj_ref, ed_ref):
    hg = jnp.maximum(_mm(x2_ref[...], wgt_ref[...]) + bg_ref[...], 0.0)
    cat = jnp.concatenate([hg, hcat_ref[...]], axis=1)
    nd = jnp.maximum(_mm(cat, wnt_ref[...]) + bn_ref[...], 0.0)
    node_ref[...] = nd
    proj_ref[...] = _mm(nd, wjt_ref[...])
    ed_ref[...] = e0_ref[...] + e1_ref[...]


def _enc(x2, hcat, e0, e1, wgt, bg, wnt, bn, wjt):
    return pl.pallas_call(
        _enc_body,
        grid=(NB,),
        in_specs=[
            pl.BlockSpec((BLK, 32), lambda i: (i, 0)),
            pl.BlockSpec((BLK, D), lambda i: (i, 0)),
            pl.BlockSpec((BLK, D), lambda i: (i, 0)),
            pl.BlockSpec((BLK, D), lambda i: (i, 0)),
            pl.BlockSpec((32, D), lambda i: (0, 0)),
            pl.BlockSpec((1, D), lambda i: (0, 0)),
            pl.BlockSpec((2 * D, D), lambda i: (0, 0)),
            pl.BlockSpec((1, D), lambda i: (0, 0)),
            pl.BlockSpec((D, D), lambda i: (0, 0)),
        ],
        out_specs=[
            pl.BlockSpec((BLK, D), lambda i: (i, 0)),
            pl.BlockSpec((BLK, D), lambda i: (i, 0)),
            pl.BlockSpec((BLK, D), lambda i: (i, 0)),
        ],
        out_shape=[
            jax.ShapeDtypeStruct((N, D), jnp.float32),
            jax.ShapeDtypeStruct((N, D), jnp.float32),
            jax.ShapeDtypeStruct((N, D), jnp.float32),
        ])(x2, hcat, e0, e1, wgt, bg, wnt, bn, wjt)


def _post_body(node_ref, s0_ref, s1_ref, ed_ref, wit_ref, bm_ref, we2t_ref,
               wiht_ref, bih_ref, whht_ref, bhh_ref, h_ref, stats_ref):
    nd = node_ref[...]
    deg = ed_ref[:, ED:ED + 1]
    aggr = (deg * (_mm(nd, wit_ref[...]) + bm_ref[...])
            + s0_ref[...] + s1_ref[...]
            + _mm(ed_ref[:, 0:ED], we2t_ref[...]))
    gi = _mm(aggr, wiht_ref[...]) + bih_ref[...]
    gh = _mm(nd, whht_ref[...]) + bhh_ref[...]
    r = jax.nn.sigmoid(gi[:, 0:D] + gh[:, 0:D])
    z = jax.nn.sigmoid(gi[:, D:2 * D] + gh[:, D:2 * D])
    n = jnp.tanh(gi[:, 2 * D:] + r * gh[:, 2 * D:])
    h = (1.0 - z) * n + z * nd
    h_ref[...] = h
    hs = jnp.sum(h, axis=0)
    h2s = jnp.sum(h * h, axis=0)
    upd = jnp.concatenate(
        [hs[None, :], h2s[None, :], jnp.zeros((6, D), jnp.float32)], axis=0)

    @pl.when(pl.program_id(0) == 0)
    def _():
        stats_ref[...] = jnp.zeros((8, D), jnp.float32)

    stats_ref[...] += upd


def _post(node, s0, s1, ed, wit, bm, we2t, wiht, bih, whht, bhh):
    return pl.pallas_call(
        _post_body,
        grid=(NB,),
        in_specs=[
            pl.BlockSpec((BLK, D), lambda i: (i, 0)),
            pl.BlockSpec((BLK, D), lambda i: (i, 0)),
            pl.BlockSpec((BLK, D), lambda i: (i, 0)),
            pl.BlockSpec((BLK, D), lambda i: (i, 0)),
            pl.BlockSpec((D, D), lambda i: (0, 0)),
            pl.BlockSpec((1, D), lambda i: (0, 0)),
            pl.BlockSpec((ED, D), lambda i: (0, 0)),
            pl.BlockSpec((D, 3 * D), lambda i: (0, 0)),
            pl.BlockSpec((1, 3 * D), lambda i: (0, 0)),
            pl.BlockSpec((D, 3 * D), lambda i: (0, 0)),
            pl.BlockSpec((1, 3 * D), lambda i: (0, 0)),
        ],
        out_specs=[
            pl.BlockSpec((BLK, D), lambda i: (i, 0)),
            pl.BlockSpec((8, D), lambda i: (0, 0)),
        ],
        out_shape=[
            jax.ShapeDtypeStruct((N, D), jnp.float32),
            jax.ShapeDtypeStruct((8, D), jnp.float32),
        ])(node, s0, s1, ed, wit, bm, we2t, wiht, bih, whht, bhh)


def _norm_common(h_ref, stats_ref, gamma_ref, beta_ref):
    mean = stats_ref[0:1, :] * (1.0 / N)
    ex2 = stats_ref[1:2, :] * (1.0 / N)
    var = ex2 - mean * mean
    return ((h_ref[...] - mean) * lax.rsqrt(var + 1e-5)
            * gamma_ref[...] + beta_ref[...])


def _norm_proj_body(h_ref, stats_ref, gamma_ref, beta_ref, wjt_ref,
                    node_ref, proj_ref):
    nd = _norm_common(h_ref, stats_ref, gamma_ref, beta_ref)
    node_ref[...] = nd
    proj_ref[...] = _mm(nd, wjt_ref[...])


def _norm_proj(h, stats, gamma, beta, wjt):
    return pl.pallas_call(
        _norm_proj_body,
        grid=(NB,),
        in_specs=[
            pl.BlockSpec((BLK, D), lambda i: (i, 0)),
            pl.BlockSpec((8, D), lambda i: (0, 0)),
            pl.BlockSpec((1, D), lambda i: (0, 0)),
            pl.BlockSpec((1, D), lambda i: (0, 0)),
            pl.BlockSpec((D, D), lambda i: (0, 0)),
        ],
        out_specs=[
            pl.BlockSpec((BLK, D), lambda i: (i, 0)),
            pl.BlockSpec((BLK, D), lambda i: (i, 0)),
        ],
        out_shape=[
            jax.ShapeDtypeStruct((N, D), jnp.float32),
            jax.ShapeDtypeStruct((N, D), jnp.float32),
        ])(h, stats, gamma, beta, wjt)


def _norm_last_body(h_ref, stats_ref, gamma_ref, beta_ref, node_ref):
    node_ref[...] = _norm_common(h_ref, stats_ref, gamma_ref, beta_ref)


def _norm_last(h, stats, gamma, beta):
    return pl.pallas_call(
        _norm_last_body,
        grid=(NB,),
        in_specs=[
            pl.BlockSpec((BLK, D), lambda i: (i, 0)),
            pl.BlockSpec((8, D), lambda i: (0, 0)),
            pl.BlockSpec((1, D), lambda i: (0, 0)),
            pl.BlockSpec((1, D), lambda i: (0, 0)),
        ],
        out_specs=pl.BlockSpec((BLK, D), lambda i: (i, 0)),
        out_shape=jax.ShapeDtypeStruct((N, D), jnp.float32))(
            h, stats, gamma, beta)


def _agg_body(node_ref, batch_ref, wlt_ref, bl_ref, wg2t_ref, bg2_ref,
              acc_ref):
    nd = node_ref[...]
    st = _mm(nd, wlt_ref[...]) + bl_ref[...]
    gz = _mm(nd, wg2t_ref[...]) + bg2_ref[...]
    m = jnp.max(gz, axis=1, keepdims=True)
    e = jnp.exp(gz - m)
    prob = e / jnp.sum(e, axis=1, keepdims=True)
    s = st * prob
    bt = batch_ref[0, 0, :]
    oh = (lax.broadcasted_iota(jnp.int32, (G, BLK), 0)
          == bt[None, :]).astype(jnp.float32)
    ones_col = (lax.broadcasted_iota(jnp.int32, (BLK, D), 1)
                == 0).astype(jnp.float32)
    sext = jnp.concatenate([s, ones_col], axis=1)
    upd = _mm(oh, sext)

    @pl.when(pl.program_id(0) == 0)
    def _():
        acc_ref[...] = jnp.zeros((G, 2 * D), jnp.float32)

    acc_ref[...] += upd


def _agg(node, batch3, wlt, bl, wg2t, bg2):
    return pl.pallas_call(
        _agg_body,
        grid=(NB,),
        in_specs=[
            pl.BlockSpec((BLK, D), lambda i: (i, 0)),
            pl.BlockSpec((1, 1, BLK), lambda i: (i, 0, 0)),
            pl.BlockSpec((D, D), lambda i: (0, 0)),
            pl.BlockSpec((1, D), lambda i: (0, 0)),
            pl.BlockSpec((D, D), lambda i: (0, 0)),
            pl.BlockSpec((1, D), lambda i: (0, 0)),
        ],
        out_specs=pl.BlockSpec((G, 2 * D), lambda i: (0, 0)),
        out_shape=jax.ShapeDtypeStruct((G, 2 * D), jnp.float32))(
            node, batch3, wlt, bl, wg2t, bg2)


def _fin_body(acc_ref, wft_ref, bf_ref, out_ref):
    summed = acc_ref[:, 0:D]
    cnt = jnp.maximum(acc_ref[:, D:D + 1], 1.0)
    out_ref[...] = _mm(summed / cnt, wft_ref[...]) + bf_ref[...]


def _fin(acc, wft, bf):
    return pl.pallas_call(
        _fin_body,
        out_shape=jax.ShapeDtypeStruct((G, D), jnp.float32))(acc, wft, bf)


def kernel(edge_index, x1, x2, edge_feats, batch, params):
    p = params
    src = edge_index[0]
    dst = edge_index[1]

    def padcore(a, padval):
        halves = []
        for ci in range(NC):
            h = a[ci * EPC:(ci + 1) * EPC]
            pad = jnp.full((EPC_PAD - EPC,) + a.shape[1:], padval, a.dtype)
            halves.append(jnp.concatenate([h, pad], axis=0))
        return jnp.stack(halves)

    srcp = padcore(src, 0)
    dstp = padcore(dst, N)
    efp = padcore(edge_feats, 0.0)
    x1p = jnp.concatenate(
        [x1[:, 0], jnp.zeros((NPAD - N,), jnp.int32)])
    batch3 = batch.reshape(NB, 1, BLK)

    wnt = p["node"]["w"].T
    wgt = p["geom"]["w"].T
    bg = p["geom"]["b"].reshape(1, D)
    bn = p["node"]["b"].reshape(1, D)
    wet = p["edge"]["w"].T
    be = p["edge"]["b"].reshape(1, ED)

    lw = []
    for lp in p["layers"]:
        wm = lp["msg"]["w"]
        lw.append(dict(
            wit=wm[:, 0:D].T,
            wjt=wm[:, D:2 * D].T,
            we2t=wm[:, 2 * D:].T,
            bm=lp["msg"]["b"].reshape(1, D),
            wiht=lp["W_ih"].T,
            bih=lp["b_ih"].reshape(1, 3 * D),
            whht=lp["W_hh"].T,
            bhh=lp["b_hh"].reshape(1, 3 * D),
            gamma=lp["gamma"].reshape(1, D),
            beta=lp["beta"].reshape(1, D),
        ))

    t_table = _tmat(p["W_cat"])
    eatt_aug = _eatt(efp.reshape(NC * EPC_PAD, ED), wet, be)
    eagg_p, hcat = _sc_pre(
        eatt_aug.reshape(NC, EPC_PAD, D), dstp, t_table, x1p)

    node, proj, ed = _enc(
        x2, hcat[:N], eagg_p[0], eagg_p[1],
        wgt, bg, wnt, bn, lw[0]["wjt"])

    for li, w in enumerate(lw):
        sp = _sc_spmv(proj, srcp, dstp)
        h, stats = _post(node, sp[0], sp[1], ed, w["wit"], w["bm"],
                         w["we2t"], w["wiht"], w["bih"], w["whht"], w["bhh"])
        if li + 1 < len(lw):
            node, proj = _norm_proj(h, stats, w["gamma"], w["beta"],
                                    lw[li + 1]["wjt"])
        else:
            node = _norm_last(h, stats, w["gamma"], w["beta"])

    acc = _agg(node, batch3,
               p["agg_lin"]["w"].T, p["agg_lin"]["b"].reshape(1, D),
               p["agg_gate"]["w"].T, p["agg_gate"]["b"].reshape(1, D))
    graph = _fin(acc, p["agg_final"]["w"].T, p["agg_final"]["b"].reshape(1, D))
    return (node, graph)


# speedup vs baseline: 3.5548x; 1.1354x over previous
import functools

import jax
import jax.numpy as jnp
from jax import lax
from jax.experimental import pallas as pl
from jax.experimental.pallas import tpu as pltpu
from jax.experimental.pallas import tpu_sc as plsc

N = 10000
E = 320000
D = 128
ED = 16
CATS = 512
G = 32

NC = 2
NS = 16
CH = 128
CPW = 80
EPC = E // NC
EPC_PAD = NS * CPW * CH
NPAD = 10240
RPW = NPAD // NS
OPW = NPAD // NS
HB = 80
HCH = 4
NBUF = 2
IH = 2
CPH = CPW // IH
BLK = 1000
NB = N // BLK

def _mm(a, b):
    return lax.dot_general(
        a, b, (((1,), (0,)), ((), ())),
        preferred_element_type=jnp.float32)


@functools.cache
def _sc_kernels():
    mesh = plsc.VectorSubcoreMesh(core_axis_name="c", subcore_axis_name="s",
                                  num_cores=NC, num_subcores=NS)

    @functools.partial(
        pl.kernel,
        out_type=[
            jax.ShapeDtypeStruct((NC, NPAD, D), jnp.float32),
            jax.ShapeDtypeStruct((NPAD, D), jnp.float32),
        ],
        mesh=mesh,
        scratch_types=[
            pltpu.VMEM((NBUF, CH, D), jnp.float32),
            pltpu.VMEM((CPH, CH), jnp.int32),
            pltpu.VMEM((HB,), jnp.int32),
            pltpu.VMEM((HB, D), jnp.float32),
            pltpu.VMEM_SHARED((NPAD, D), jnp.float32),
            pltpu.SemaphoreType.DMA,
            pltpu.SemaphoreType.DMA,
        ])
    def sc_pre(eatt_hbm, dst_hbm, t_hbm, x1_hbm, eagg_out, hcat_out,
               ebuf, dbuf, xbuf, rbuf, accum, sem0, sem1):
        sems = (sem0, sem1)
        c = lax.axis_index("c")
        s = lax.axis_index("s")
        wid = c * NS + s

        @pl.loop(0, CH)
        def _(r):
            @pl.loop(0, D // 16)
            def _(j):
                ebuf[0, r, pl.ds(j * 16, 16)] = jnp.zeros((16,), jnp.float32)

        @pl.loop(0, RPW // CH)
        def _(k):
            pltpu.sync_copy(ebuf.at[0], accum.at[pl.ds(s * RPW + k * CH, CH)])

        plsc.subcore_barrier()

        @pl.loop(0, IH)
        def _(hf):
            pltpu.sync_copy(dst_hbm.at[c].at[s].at[pl.ds(hf * CPH, CPH)],
                            dbuf)

            @pl.loop(0, CPH // NBUF)
            def _(g):
                for b in range(NBUF):
                    k = hf * CPH + g * NBUF + b
                    pltpu.async_copy(
                        eatt_hbm.at[c].at[pl.ds((s * CPW + k) * CH, CH)],
                        ebuf.at[b], sems[b])
                for b in range(NBUF):
                    k = hf * CPH + g * NBUF + b
                    pltpu.make_async_copy(
                        eatt_hbm.at[c].at[pl.ds((s * CPW + k) * CH, CH)],
                        ebuf.at[b], sems[b]).wait()
                    pltpu.sync_copy(ebuf.at[b],
                                    accum.at[dbuf.at[g * NBUF + b]],
                                    add=True)

        plsc.subcore_barrier()
        pltpu.sync_copy(accum.at[pl.ds(s * OPW, OPW)],
                        eagg_out.at[c].at[pl.ds(s * OPW, OPW)])

        @pl.loop(0, HCH)
        def _(k):
            b = wid * (HCH * HB) + k * HB
            pltpu.sync_copy(x1_hbm.at[pl.ds(b, HB)], xbuf)
            pltpu.async_copy(t_hbm.at[xbuf], rbuf, sem0).wait()
            pltpu.sync_copy(rbuf, hcat_out.at[pl.ds(b, HB)])

    @functools.partial(
        pl.kernel,
        out_type=jax.ShapeDtypeStruct((NC, NPAD, D), jnp.float32),
        mesh=mesh,
        scratch_types=[
            pltpu.VMEM((NBUF, CH, D), jnp.float32),
            pltpu.VMEM((CPH, CH), jnp.int32),
            pltpu.VMEM((CPH, CH), jnp.int32),
            pltpu.VMEM_SHARED((NPAD, D), jnp.float32),
            pltpu.SemaphoreType.DMA,
            pltpu.SemaphoreType.DMA,
        ])
    def sc_spmv(p_hbm, src_hbm, dst_hbm, out_hbm, rbuf, sbuf, dbuf,
                accum, sem0, sem1):
        sems = (sem0, sem1)
        c = lax.axis_index("c")
        s = lax.axis_index("s")

        @pl.loop(0, CH)
        def _(r):
            @pl.loop(0, D // 16)
            def _(j):
                rbuf[0, r, pl.ds(j * 16, 16)] = jnp.zeros((16,), jnp.float32)

        @pl.loop(0, RPW // CH)
        def _(k):
            pltpu.sync_copy(rbuf.at[0], accum.at[pl.ds(s * RPW + k * CH, CH)])

        plsc.subcore_barrier()

        @pl.loop(0, IH)
        def _(hf):
            pltpu.sync_copy(src_hbm.at[c].at[s].at[pl.ds(hf * CPH, CPH)],
                            sbuf)
            pltpu.sync_copy(dst_hbm.at[c].at[s].at[pl.ds(hf * CPH, CPH)],
                            dbuf)

            @pl.loop(0, CPH // NBUF)
            def _(g):
                for b in range(NBUF):
                    pltpu.async_copy(p_hbm.at[sbuf.at[g * NBUF + b]],
                                     rbuf.at[b], sems[b])
                for b in range(NBUF):
                    pltpu.make_async_copy(p_hbm.at[sbuf.at[g * NBUF + b]],
                                          rbuf.at[b], sems[b]).wait()
                    pltpu.sync_copy(rbuf.at[b],
                                    accum.at[dbuf.at[g * NBUF + b]],
                                    add=True)

        plsc.subcore_barrier()
        pltpu.sync_copy(accum.at[pl.ds(s * OPW, OPW)],
                        out_hbm.at[c].at[pl.ds(s * OPW, OPW)])

    return sc_pre, sc_spmv


def _sc_pre(eatt3, dstp, t_table, x1p):
    return _sc_kernels()[0](eatt3, dstp, t_table, x1p)


def _sc_spmv(proj, srcp, dstp):
    return _sc_kernels()[1](proj, srcp, dstp)


def _tmat_body(wcat_ref, out_ref):
    out_ref[...] = jnp.maximum(wcat_ref[...], 0.0)


def _tmat(wcat):
    return pl.pallas_call(
        _tmat_body,
        out_shape=jax.ShapeDtypeStruct((CATS, D), jnp.float32))(wcat)


_EB = 4096


def _eatt_body(ef_ref, wet_ref, be_ref, out_ref):
    x = jnp.maximum(_mm(ef_ref[...], wet_ref[...]) + be_ref[...], 0.0)
    pad = (lax.broadcasted_iota(jnp.int32, (_EB, D - ED), 1) == 0)
    out_ref[...] = jnp.concatenate([x, pad.astype(jnp.float32)], axis=1)


def _eatt(ef_pad, wet, be):
    ne = ef_pad.shape[0]
    return pl.pallas_call(
        _eatt_body,
        grid=(ne // _EB,),
        in_specs=[
            pl.BlockSpec((_EB, ED), lambda i: (i, 0)),
            pl.BlockSpec((ED, ED), lambda i: (0, 0)),
            pl.BlockSpec((1, ED), lambda i: (0, 0)),
        ],
        out_specs=pl.BlockSpec((_EB, D), lambda i: (i, 0)),
        out_shape=jax.ShapeDtypeStruct((ne, D), jnp.float32))(
            ef_pad, wet, be)


def _enc_body(x2_ref, hcat_ref, e0_ref, e1_ref, wgt_ref, bg_ref, wnt_ref,
              bn_ref, wjt_ref, node_ref, proj_ref, ed_ref):
    hg = jnp.maximum(_mm(x2_ref[...], wgt_ref[...]) + bg_ref[...], 0.0)
    cat = jnp.concatenate([hg, hcat_ref[...]], axis=1)
    nd = jnp.maximum(_mm(cat, wnt_ref[...]) + bn_ref[...], 0.0)
    node_ref[...] = nd
    proj_ref[...] = _mm(nd, wjt_ref[...])
    ed_ref[...] = e0_ref[...] + e1_ref[...]


def _enc(x2, hcat, e0, e1, wgt, bg, wnt, bn, wjt):
    return pl.pallas_call(
        _enc_body,
        grid=(NB,),
        in_specs=[
            pl.BlockSpec((BLK, 32), lambda i: (i, 0)),
            pl.BlockSpec((BLK, D), lambda i: (i, 0)),
            pl.BlockSpec((BLK, D), lambda i: (i, 0)),
            pl.BlockSpec((BLK, D), lambda i: (i, 0)),
            pl.BlockSpec((32, D), lambda i: (0, 0)),
            pl.BlockSpec((1, D), lambda i: (0, 0)),
            pl.BlockSpec((2 * D, D), lambda i: (0, 0)),
            pl.BlockSpec((1, D), lambda i: (0, 0)),
            pl.BlockSpec((D, D), lambda i: (0, 0)),
        ],
        out_specs=[
            pl.BlockSpec((BLK, D), lambda i: (i, 0)),
            pl.BlockSpec((BLK, D), lambda i: (i, 0)),
            pl.BlockSpec((BLK, D), lambda i: (i, 0)),
        ],
        out_shape=[
            jax.ShapeDtypeStruct((N, D), jnp.float32),
            jax.ShapeDtypeStruct((N, D), jnp.float32),
            jax.ShapeDtypeStruct((N, D), jnp.float32),
        ])(x2, hcat, e0, e1, wgt, bg, wnt, bn, wjt)


def _post_body(node_ref, s0_ref, s1_ref, ed_ref, wit_ref, bm_ref, we2t_ref,
               wiht_ref, bih_ref, whht_ref, bhh_ref, h_ref, stats_ref):
    nd = node_ref[...]
    deg = ed_ref[:, ED:ED + 1]
    aggr = (deg * (_mm(nd, wit_ref[...]) + bm_ref[...])
            + s0_ref[...] + s1_ref[...]
            + _mm(ed_ref[:, 0:ED], we2t_ref[...]))
    gi = _mm(aggr, wiht_ref[...]) + bih_ref[...]
    gh = _mm(nd, whht_ref[...]) + bhh_ref[...]
    r = jax.nn.sigmoid(gi[:, 0:D] + gh[:, 0:D])
    z = jax.nn.sigmoid(gi[:, D:2 * D] + gh[:, D:2 * D])
    n = jnp.tanh(gi[:, 2 * D:] + r * gh[:, 2 * D:])
    h = (1.0 - z) * n + z * nd
    h_ref[...] = h
    hs = jnp.sum(h, axis=0)
    h2s = jnp.sum(h * h, axis=0)
    upd = jnp.concatenate(
        [hs[None, :], h2s[None, :], jnp.zeros((6, D), jnp.float32)], axis=0)

    @pl.when(pl.program_id(0) == 0)
    def _():
        stats_ref[...] = jnp.zeros((8, D), jnp.float32)

    stats_ref[...] += upd


def _post(node, s0, s1, ed, wit, bm, we2t, wiht, bih, whht, bhh):
    return pl.pallas_call(
        _post_body,
        grid=(NB,),
        in_specs=[
            pl.BlockSpec((BLK, D), lambda i: (i, 0)),
            pl.BlockSpec((BLK, D), lambda i: (i, 0)),
            pl.BlockSpec((BLK, D), lambda i: (i, 0)),
            pl.BlockSpec((BLK, D), lambda i: (i, 0)),
            pl.BlockSpec((D, D), lambda i: (0, 0)),
            pl.BlockSpec((1, D), lambda i: (0, 0)),
            pl.BlockSpec((ED, D), lambda i: (0, 0)),
            pl.BlockSpec((D, 3 * D), lambda i: (0, 0)),
            pl.BlockSpec((1, 3 * D), lambda i: (0, 0)),
            pl.BlockSpec((D, 3 * D), lambda i: (0, 0)),
            pl.BlockSpec((1, 3 * D), lambda i: (0, 0)),
        ],
        out_specs=[
            pl.BlockSpec((BLK, D), lambda i: (i, 0)),
            pl.BlockSpec((8, D), lambda i: (0, 0)),
        ],
        out_shape=[
            jax.ShapeDtypeStruct((N, D), jnp.float32),
            jax.ShapeDtypeStruct((8, D), jnp.float32),
        ])(node, s0, s1, ed, wit, bm, we2t, wiht, bih, whht, bhh)


def _norm_common(h_ref, stats_ref, gamma_ref, beta_ref):
    mean = stats_ref[0:1, :] * (1.0 / N)
    ex2 = stats_ref[1:2, :] * (1.0 / N)
    var = ex2 - mean * mean
    return ((h_ref[...] - mean) * lax.rsqrt(var + 1e-5)
            * gamma_ref[...] + beta_ref[...])


def _norm_proj_body(h_ref, stats_ref, gamma_ref, beta_ref, wjt_ref,
                    node_ref, proj_ref):
    nd = _norm_common(h_ref, stats_ref, gamma_ref, beta_ref)
    node_ref[...] = nd
    proj_ref[...] = _mm(nd, wjt_ref[...])


def _norm_proj(h, stats, gamma, beta, wjt):
    return pl.pallas_call(
        _norm_proj_body,
        grid=(NB,),
        in_specs=[
            pl.BlockSpec((BLK, D), lambda i: (i, 0)),
            pl.BlockSpec((8, D), lambda i: (0, 0)),
            pl.BlockSpec((1, D), lambda i: (0, 0)),
            pl.BlockSpec((1, D), lambda i: (0, 0)),
            pl.BlockSpec((D, D), lambda i: (0, 0)),
        ],
        out_specs=[
            pl.BlockSpec((BLK, D), lambda i: (i, 0)),
            pl.BlockSpec((BLK, D), lambda i: (i, 0)),
        ],
        out_shape=[
            jax.ShapeDtypeStruct((N, D), jnp.float32),
            jax.ShapeDtypeStruct((N, D), jnp.float32),
        ])(h, stats, gamma, beta, wjt)


def _norm_last_body(h_ref, stats_ref, gamma_ref, beta_ref, node_ref):
    node_ref[...] = _norm_common(h_ref, stats_ref, gamma_ref, beta_ref)


def _norm_last(h, stats, gamma, beta):
    return pl.pallas_call(
        _norm_last_body,
        grid=(NB,),
        in_specs=[
            pl.BlockSpec((BLK, D), lambda i: (i, 0)),
            pl.BlockSpec((8, D), lambda i: (0, 0)),
            pl.BlockSpec((1, D), lambda i: (0, 0)),
            pl.BlockSpec((1, D), lambda i: (0, 0)),
        ],
        out_specs=pl.BlockSpec((BLK, D), lambda i: (i, 0)),
        out_shape=jax.ShapeDtypeStruct((N, D), jnp.float32))(
            h, stats, gamma, beta)


def _agg_body(node_ref, batch_ref, wlt_ref, bl_ref, wg2t_ref, bg2_ref,
              acc_ref):
    nd = node_ref[...]
    st = _mm(nd, wlt_ref[...]) + bl_ref[...]
    gz = _mm(nd, wg2t_ref[...]) + bg2_ref[...]
    m = jnp.max(gz, axis=1, keepdims=True)
    e = jnp.exp(gz - m)
    prob = e / jnp.sum(e, axis=1, keepdims=True)
    s = st * prob
    bt = batch_ref[0, 0, :]
    oh = (lax.broadcasted_iota(jnp.int32, (G, BLK), 0)
          == bt[None, :]).astype(jnp.float32)
    ones_col = (lax.broadcasted_iota(jnp.int32, (BLK, D), 1)
                == 0).astype(jnp.float32)
    sext = jnp.concatenate([s, ones_col], axis=1)
    upd = _mm(oh, sext)

    @pl.when(pl.program_id(0) == 0)
    def _():
        acc_ref[...] = jnp.zeros((G, 2 * D), jnp.float32)

    acc_ref[...] += upd


def _agg(node, batch3, wlt, bl, wg2t, bg2):
    return pl.pallas_call(
        _agg_body,
        grid=(NB,),
        in_specs=[
            pl.BlockSpec((BLK, D), lambda i: (i, 0)),
            pl.BlockSpec((1, 1, BLK), lambda i: (i, 0, 0)),
            pl.BlockSpec((D, D), lambda i: (0, 0)),
            pl.BlockSpec((1, D), lambda i: (0, 0)),
            pl.BlockSpec((D, D), lambda i: (0, 0)),
            pl.BlockSpec((1, D), lambda i: (0, 0)),
        ],
        out_specs=pl.BlockSpec((G, 2 * D), lambda i: (0, 0)),
        out_shape=jax.ShapeDtypeStruct((G, 2 * D), jnp.float32))(
            node, batch3, wlt, bl, wg2t, bg2)


def _fin_body(acc_ref, wft_ref, bf_ref, out_ref):
    summed = acc_ref[:, 0:D]
    cnt = jnp.maximum(acc_ref[:, D:D + 1], 1.0)
    out_ref[...] = _mm(summed / cnt, wft_ref[...]) + bf_ref[...]


def _fin(acc, wft, bf):
    return pl.pallas_call(
        _fin_body,
        out_shape=jax.ShapeDtypeStruct((G, D), jnp.float32))(acc, wft, bf)


def kernel(edge_index, x1, x2, edge_feats, batch, params):
    p = params
    src = edge_index[0]
    dst = edge_index[1]

    def padcore(a, padval):
        halves = []
        for ci in range(NC):
            h = a[ci * EPC:(ci + 1) * EPC]
            pad = jnp.full((EPC_PAD - EPC,) + a.shape[1:], padval, a.dtype)
            halves.append(jnp.concatenate([h, pad], axis=0))
        return jnp.stack(halves)

    srcp = padcore(src, 0).reshape(NC, NS, CPW, CH)
    dstp = padcore(dst, N)
    dstp4 = dstp.reshape(NC, NS, CPW, CH)
    efp = padcore(edge_feats, 0.0)
    x1p = jnp.concatenate(
        [x1[:, 0], jnp.zeros((NPAD - N,), jnp.int32)])
    batch3 = batch.reshape(NB, 1, BLK)

    wnt = p["node"]["w"].T
    wgt = p["geom"]["w"].T
    bg = p["geom"]["b"].reshape(1, D)
    bn = p["node"]["b"].reshape(1, D)
    wet = p["edge"]["w"].T
    be = p["edge"]["b"].reshape(1, ED)

    lw = []
    for lp in p["layers"]:
        wm = lp["msg"]["w"]
        lw.append(dict(
            wit=wm[:, 0:D].T,
            wjt=wm[:, D:2 * D].T,
            we2t=wm[:, 2 * D:].T,
            bm=lp["msg"]["b"].reshape(1, D),
            wiht=lp["W_ih"].T,
            bih=lp["b_ih"].reshape(1, 3 * D),
            whht=lp["W_hh"].T,
            bhh=lp["b_hh"].reshape(1, 3 * D),
            gamma=lp["gamma"].reshape(1, D),
            beta=lp["beta"].reshape(1, D),
        ))

    t_table = _tmat(p["W_cat"])
    eatt_aug = _eatt(efp.reshape(NC * EPC_PAD, ED), wet, be)
    eagg_p, hcat = _sc_pre(
        eatt_aug.reshape(NC, EPC_PAD, D), dstp4, t_table, x1p)

    node, proj, ed = _enc(
        x2, hcat[:N], eagg_p[0], eagg_p[1],
        wgt, bg, wnt, bn, lw[0]["wjt"])

    for li, w in enumerate(lw):
        sp = _sc_spmv(proj, srcp, dstp4)
        h, stats = _post(node, sp[0], sp[1], ed, w["wit"], w["bm"],
                         w["we2t"], w["wiht"], w["bih"], w["whht"], w["bhh"])
        if li + 1 < len(lw):
            node, proj = _norm_proj(h, stats, w["gamma"], w["beta"],
                                    lw[li + 1]["wjt"])
        else:
            node = _norm_last(h, stats, w["gamma"], w["beta"])

    acc = _agg(node, batch3,
               p["agg_lin"]["w"].T, p["agg_lin"]["b"].reshape(1, D),
               p["agg_gate"]["w"].T, p["agg_gate"]["b"].reshape(1, D))
    graph = _fin(acc, p["agg_final"]["w"].T, p["agg_final"]["b"].reshape(1, D))
    return (node, graph)


# speedup vs baseline: 3.6638x; 1.0307x over previous
import functools

import jax
import jax.numpy as jnp
from jax import lax
from jax.experimental import pallas as pl
from jax.experimental.pallas import tpu as pltpu
from jax.experimental.pallas import tpu_sc as plsc

N = 10000
E = 320000
D = 128
ED = 16
CATS = 512
G = 32

NC = 2
NS = 16
CH = 128
CPW = 80
EPC = E // NC
EPC_PAD = NS * CPW * CH
NPAD = 10240
RPW = NPAD // NS
OPW = NPAD // NS
HB = 80
HCH = 4
NBUF = 2
IH = 2
CPH = CPW // IH
BLK = 1000
NB = N // BLK

def _mm(a, b):
    return lax.dot_general(
        a, b, (((1,), (0,)), ((), ())),
        preferred_element_type=jnp.float32)


def _mmh(a, b):
    return lax.dot_general(
        a, b, (((1,), (0,)), ((), ())),
        precision=lax.Precision.HIGHEST,
        preferred_element_type=jnp.float32)


@functools.cache
def _sc_kernels():
    mesh = plsc.VectorSubcoreMesh(core_axis_name="c", subcore_axis_name="s",
                                  num_cores=NC, num_subcores=NS)

    @functools.partial(
        pl.kernel,
        out_type=[
            jax.ShapeDtypeStruct((NC, NPAD, D), jnp.float32),
            jax.ShapeDtypeStruct((NPAD, D), jnp.float32),
        ],
        mesh=mesh,
        scratch_types=[
            pltpu.VMEM((NBUF, CH, D), jnp.float32),
            pltpu.VMEM((CPH, CH), jnp.int32),
            pltpu.VMEM((HB,), jnp.int32),
            pltpu.VMEM((HB, D), jnp.float32),
            pltpu.VMEM_SHARED((NPAD, D), jnp.float32),
            pltpu.SemaphoreType.DMA,
            pltpu.SemaphoreType.DMA,
            pltpu.SemaphoreType.DMA,
            pltpu.SemaphoreType.DMA,
        ])
    def sc_pre(eatt_hbm, dst_hbm, t_hbm, x1_hbm, eagg_out, hcat_out,
               ebuf, dbuf, xbuf, rbuf, accum, gs0, gs1, ss0, ss1):
        gsems = (gs0, gs1)
        ssems = (ss0, ss1)
        c = lax.axis_index("c")
        s = lax.axis_index("s")
        wid = c * NS + s

        @pl.loop(0, CH)
        def _(r):
            @pl.loop(0, D // 16)
            def _(j):
                ebuf[0, r, pl.ds(j * 16, 16)] = jnp.zeros((16,), jnp.float32)

        @pl.loop(0, RPW // CH)
        def _(k):
            pltpu.sync_copy(ebuf.at[0], accum.at[pl.ds(s * RPW + k * CH, CH)])

        plsc.subcore_barrier()

        @pl.loop(0, IH)
        def _(hf):
            pltpu.sync_copy(dst_hbm.at[c].at[s].at[pl.ds(hf * CPH, CPH)],
                            dbuf)
            for b in range(NBUF):
                k = hf * CPH + b
                pltpu.async_copy(
                    eatt_hbm.at[c].at[pl.ds((s * CPW + k) * CH, CH)],
                    ebuf.at[b], gsems[b])

            @pl.loop(0, CPH // NBUF)
            def _(g):
                for b in range(NBUF):
                    k = hf * CPH + g * NBUF + b
                    pltpu.make_async_copy(
                        eatt_hbm.at[c].at[pl.ds((s * CPW + k) * CH, CH)],
                        ebuf.at[b], gsems[b]).wait()
                    pltpu.async_copy(ebuf.at[b],
                                     accum.at[dbuf.at[g * NBUF + b]],
                                     ssems[b], add=True)
                for b in range(NBUF):
                    k = g * NBUF + b

                    @pl.when(k + NBUF < CPH)
                    def _():
                        pltpu.make_async_copy(ebuf.at[b],
                                              accum.at[dbuf.at[k]],
                                              ssems[b]).wait()
                        pltpu.async_copy(
                            eatt_hbm.at[c].at[
                                pl.ds((s * CPW + hf * CPH + k + NBUF) * CH,
                                      CH)],
                            ebuf.at[b], gsems[b])

            for b in range(NBUF):
                pltpu.make_async_copy(ebuf.at[b],
                                      accum.at[dbuf.at[CPH - NBUF + b]],
                                      ssems[b]).wait()

        plsc.subcore_barrier()
        pltpu.sync_copy(accum.at[pl.ds(s * OPW, OPW)],
                        eagg_out.at[c].at[pl.ds(s * OPW, OPW)])

        @pl.loop(0, HCH)
        def _(k):
            b = wid * (HCH * HB) + k * HB
            pltpu.sync_copy(x1_hbm.at[pl.ds(b, HB)], xbuf)
            pltpu.async_copy(t_hbm.at[xbuf], rbuf, gs0).wait()
            pltpu.sync_copy(rbuf, hcat_out.at[pl.ds(b, HB)])

    @functools.partial(
        pl.kernel,
        out_type=jax.ShapeDtypeStruct((NC, NPAD, D), jnp.float32),
        mesh=mesh,
        scratch_types=[
            pltpu.VMEM((NBUF, CH, D), jnp.float32),
            pltpu.VMEM((CPH, CH), jnp.int32),
            pltpu.VMEM((CPH, CH), jnp.int32),
            pltpu.VMEM_SHARED((NPAD, D), jnp.float32),
            pltpu.SemaphoreType.DMA,
            pltpu.SemaphoreType.DMA,
            pltpu.SemaphoreType.DMA,
            pltpu.SemaphoreType.DMA,
        ])
    def sc_spmv(p_hbm, src_hbm, dst_hbm, out_hbm, rbuf, sbuf, dbuf,
                accum, gs0, gs1, ss0, ss1):
        gsems = (gs0, gs1)
        ssems = (ss0, ss1)
        c = lax.axis_index("c")
        s = lax.axis_index("s")

        @pl.loop(0, CH)
        def _(r):
            @pl.loop(0, D // 16)
            def _(j):
                rbuf[0, r, pl.ds(j * 16, 16)] = jnp.zeros((16,), jnp.float32)

        @pl.loop(0, RPW // CH)
        def _(k):
            pltpu.sync_copy(rbuf.at[0], accum.at[pl.ds(s * RPW + k * CH, CH)])

        plsc.subcore_barrier()

        @pl.loop(0, IH)
        def _(hf):
            pltpu.sync_copy(src_hbm.at[c].at[s].at[pl.ds(hf * CPH, CPH)],
                            sbuf)
            pltpu.sync_copy(dst_hbm.at[c].at[s].at[pl.ds(hf * CPH, CPH)],
                            dbuf)
            for b in range(NBUF):
                pltpu.async_copy(p_hbm.at[sbuf.at[b]], rbuf.at[b], gsems[b])

            @pl.loop(0, CPH // NBUF)
            def _(g):
                for b in range(NBUF):
                    k = g * NBUF + b
                    pltpu.make_async_copy(p_hbm.at[sbuf.at[k]],
                                          rbuf.at[b], gsems[b]).wait()
                    pltpu.async_copy(rbuf.at[b], accum.at[dbuf.at[k]],
                                     ssems[b], add=True)
                for b in range(NBUF):
                    k = g * NBUF + b

                    @pl.when(k + NBUF < CPH)
                    def _():
                        pltpu.make_async_copy(rbuf.at[b],
                                              accum.at[dbuf.at[k]],
                                              ssems[b]).wait()
                        pltpu.async_copy(p_hbm.at[sbuf.at[k + NBUF]],
                                         rbuf.at[b], gsems[b])

            for b in range(NBUF):
                pltpu.make_async_copy(rbuf.at[b],
                                      accum.at[dbuf.at[CPH - NBUF + b]],
                                      ssems[b]).wait()

        plsc.subcore_barrier()
        pltpu.sync_copy(accum.at[pl.ds(s * OPW, OPW)],
                        out_hbm.at[c].at[pl.ds(s * OPW, OPW)])

    return sc_pre, sc_spmv


def _sc_pre(eatt3, dstp, t_table, x1p):
    return _sc_kernels()[0](eatt3, dstp, t_table, x1p)


def _sc_spmv(proj, srcp, dstp):
    return _sc_kernels()[1](proj, srcp, dstp)


def _tmat_body(wcat_ref, out_ref):
    out_ref[...] = jnp.maximum(wcat_ref[...], 0.0)


def _tmat(wcat):
    return pl.pallas_call(
        _tmat_body,
        out_shape=jax.ShapeDtypeStruct((CATS, D), jnp.float32))(wcat)


_EB = 4096


def _eatt_body(ef_ref, wet_ref, be_ref, out_ref):
    x = jnp.maximum(_mm(ef_ref[...], wet_ref[...]) + be_ref[...], 0.0)
    pad = (lax.broadcasted_iota(jnp.int32, (_EB, D - ED), 1) == 0)
    out_ref[...] = jnp.concatenate([x, pad.astype(jnp.float32)], axis=1)


def _eatt(ef_pad, wet, be):
    ne = ef_pad.shape[0]
    return pl.pallas_call(
        _eatt_body,
        grid=(ne // _EB,),
        in_specs=[
            pl.BlockSpec((_EB, ED), lambda i: (i, 0)),
            pl.BlockSpec((ED, ED), lambda i: (0, 0)),
            pl.BlockSpec((1, ED), lambda i: (0, 0)),
        ],
        out_specs=pl.BlockSpec((_EB, D), lambda i: (i, 0)),
        out_shape=jax.ShapeDtypeStruct((ne, D), jnp.float32))(
            ef_pad, wet, be)


def _enc_body(x2_ref, hcat_ref, e0_ref, e1_ref, wgt_ref, bg_ref, wnt_ref,
              bn_ref, wjt_ref, node_ref, proj_ref, ed_ref):
    hg = jnp.maximum(_mm(x2_ref[...], wgt_ref[...]) + bg_ref[...], 0.0)
    cat = jnp.concatenate([hg, hcat_ref[...]], axis=1)
    nd = jnp.maximum(_mm(cat, wnt_ref[...]) + bn_ref[...], 0.0)
    node_ref[...] = nd
    proj_ref[...] = _mmh(nd, wjt_ref[...])
    ed_ref[...] = e0_ref[...] + e1_ref[...]


def _enc(x2, hcat, e0, e1, wgt, bg, wnt, bn, wjt):
    return pl.pallas_call(
        _enc_body,
        grid=(NB,),
        in_specs=[
            pl.BlockSpec((BLK, 32), lambda i: (i, 0)),
            pl.BlockSpec((BLK, D), lambda i: (i, 0)),
            pl.BlockSpec((BLK, D), lambda i: (i, 0)),
            pl.BlockSpec((BLK, D), lambda i: (i, 0)),
            pl.BlockSpec((32, D), lambda i: (0, 0)),
            pl.BlockSpec((1, D), lambda i: (0, 0)),
            pl.BlockSpec((2 * D, D), lambda i: (0, 0)),
            pl.BlockSpec((1, D), lambda i: (0, 0)),
            pl.BlockSpec((D, D), lambda i: (0, 0)),
        ],
        out_specs=[
            pl.BlockSpec((BLK, D), lambda i: (i, 0)),
            pl.BlockSpec((BLK, D), lambda i: (i, 0)),
            pl.BlockSpec((BLK, D), lambda i: (i, 0)),
        ],
        out_shape=[
            jax.ShapeDtypeStruct((N, D), jnp.float32),
            jax.ShapeDtypeStruct((N, D), jnp.float32),
            jax.ShapeDtypeStruct((N, D), jnp.float32),
        ])(x2, hcat, e0, e1, wgt, bg, wnt, bn, wjt)


def _post_body(node_ref, s0_ref, s1_ref, ed_ref, wit_ref, bm_ref, we2t_ref,
               wiht_ref, bih_ref, whht_ref, bhh_ref, h_ref, stats_ref):
    nd = node_ref[...]
    deg = ed_ref[:, ED:ED + 1]
    aggr = (deg * (_mmh(nd, wit_ref[...]) + bm_ref[...])
            + s0_ref[...] + s1_ref[...]
            + _mmh(ed_ref[:, 0:ED], we2t_ref[...]))
    gi = _mm(aggr, wiht_ref[...]) + bih_ref[...]
    gh = _mm(nd, whht_ref[...]) + bhh_ref[...]
    r = jax.nn.sigmoid(gi[:, 0:D] + gh[:, 0:D])
    z = jax.nn.sigmoid(gi[:, D:2 * D] + gh[:, D:2 * D])
    n = jnp.tanh(gi[:, 2 * D:] + r * gh[:, 2 * D:])
    h = (1.0 - z) * n + z * nd
    h_ref[...] = h
    hs = jnp.sum(h, axis=0)
    h2s = jnp.sum(h * h, axis=0)
    upd = jnp.concatenate(
        [hs[None, :], h2s[None, :], jnp.zeros((6, D), jnp.float32)], axis=0)

    @pl.when(pl.program_id(0) == 0)
    def _():
        stats_ref[...] = jnp.zeros((8, D), jnp.float32)

    stats_ref[...] += upd


def _post(node, s0, s1, ed, wit, bm, we2t, wiht, bih, whht, bhh):
    return pl.pallas_call(
        _post_body,
        grid=(NB,),
        in_specs=[
            pl.BlockSpec((BLK, D), lambda i: (i, 0)),
            pl.BlockSpec((BLK, D), lambda i: (i, 0)),
            pl.BlockSpec((BLK, D), lambda i: (i, 0)),
            pl.BlockSpec((BLK, D), lambda i: (i, 0)),
            pl.BlockSpec((D, D), lambda i: (0, 0)),
            pl.BlockSpec((1, D), lambda i: (0, 0)),
            pl.BlockSpec((ED, D), lambda i: (0, 0)),
            pl.BlockSpec((D, 3 * D), lambda i: (0, 0)),
            pl.BlockSpec((1, 3 * D), lambda i: (0, 0)),
            pl.BlockSpec((D, 3 * D), lambda i: (0, 0)),
            pl.BlockSpec((1, 3 * D), lambda i: (0, 0)),
        ],
        out_specs=[
            pl.BlockSpec((BLK, D), lambda i: (i, 0)),
            pl.BlockSpec((8, D), lambda i: (0, 0)),
        ],
        out_shape=[
            jax.ShapeDtypeStruct((N, D), jnp.float32),
            jax.ShapeDtypeStruct((8, D), jnp.float32),
        ])(node, s0, s1, ed, wit, bm, we2t, wiht, bih, whht, bhh)


def _norm_common(h_ref, stats_ref, gamma_ref, beta_ref):
    mean = stats_ref[0:1, :] * (1.0 / N)
    ex2 = stats_ref[1:2, :] * (1.0 / N)
    var = ex2 - mean * mean
    return ((h_ref[...] - mean) * lax.rsqrt(var + 1e-5)
            * gamma_ref[...] + beta_ref[...])


def _norm_proj_body(h_ref, stats_ref, gamma_ref, beta_ref, wjt_ref,
                    node_ref, proj_ref):
    nd = _norm_common(h_ref, stats_ref, gamma_ref, beta_ref)
    node_ref[...] = nd
    proj_ref[...] = _mmh(nd, wjt_ref[...])


def _norm_proj(h, stats, gamma, beta, wjt):
    return pl.pallas_call(
        _norm_proj_body,
        grid=(NB,),
        in_specs=[
            pl.BlockSpec((BLK, D), lambda i: (i, 0)),
            pl.BlockSpec((8, D), lambda i: (0, 0)),
            pl.BlockSpec((1, D), lambda i: (0, 0)),
            pl.BlockSpec((1, D), lambda i: (0, 0)),
            pl.BlockSpec((D, D), lambda i: (0, 0)),
        ],
        out_specs=[
            pl.BlockSpec((BLK, D), lambda i: (i, 0)),
            pl.BlockSpec((BLK, D), lambda i: (i, 0)),
        ],
        out_shape=[
            jax.ShapeDtypeStruct((N, D), jnp.float32),
            jax.ShapeDtypeStruct((N, D), jnp.float32),
        ])(h, stats, gamma, beta, wjt)


def _norm_last_body(h_ref, stats_ref, gamma_ref, beta_ref, node_ref):
    node_ref[...] = _norm_common(h_ref, stats_ref, gamma_ref, beta_ref)


def _norm_last(h, stats, gamma, beta):
    return pl.pallas_call(
        _norm_last_body,
        grid=(NB,),
        in_specs=[
            pl.BlockSpec((BLK, D), lambda i: (i, 0)),
            pl.BlockSpec((8, D), lambda i: (0, 0)),
            pl.BlockSpec((1, D), lambda i: (0, 0)),
            pl.BlockSpec((1, D), lambda i: (0, 0)),
        ],
        out_specs=pl.BlockSpec((BLK, D), lambda i: (i, 0)),
        out_shape=jax.ShapeDtypeStruct((N, D), jnp.float32))(
            h, stats, gamma, beta)


def _agg_body(node_ref, batch_ref, wlt_ref, bl_ref, wg2t_ref, bg2_ref,
              acc_ref):
    nd = node_ref[...]
    st = _mm(nd, wlt_ref[...]) + bl_ref[...]
    gz = _mm(nd, wg2t_ref[...]) + bg2_ref[...]
    m = jnp.max(gz, axis=1, keepdims=True)
    e = jnp.exp(gz - m)
    prob = e / jnp.sum(e, axis=1, keepdims=True)
    s = st * prob
    bt = batch_ref[0, 0, :]
    oh = (lax.broadcasted_iota(jnp.int32, (G, BLK), 0)
          == bt[None, :]).astype(jnp.float32)
    ones_col = (lax.broadcasted_iota(jnp.int32, (BLK, D), 1)
                == 0).astype(jnp.float32)
    sext = jnp.concatenate([s, ones_col], axis=1)
    upd = _mm(oh, sext)

    @pl.when(pl.program_id(0) == 0)
    def _():
        acc_ref[...] = jnp.zeros((G, 2 * D), jnp.float32)

    acc_ref[...] += upd


def _agg(node, batch3, wlt, bl, wg2t, bg2):
    return pl.pallas_call(
        _agg_body,
        grid=(NB,),
        in_specs=[
            pl.BlockSpec((BLK, D), lambda i: (i, 0)),
            pl.BlockSpec((1, 1, BLK), lambda i: (i, 0, 0)),
            pl.BlockSpec((D, D), lambda i: (0, 0)),
            pl.BlockSpec((1, D), lambda i: (0, 0)),
            pl.BlockSpec((D, D), lambda i: (0, 0)),
            pl.BlockSpec((1, D), lambda i: (0, 0)),
        ],
        out_specs=pl.BlockSpec((G, 2 * D), lambda i: (0, 0)),
        out_shape=jax.ShapeDtypeStruct((G, 2 * D), jnp.float32))(
            node, batch3, wlt, bl, wg2t, bg2)


def _fin_body(acc_ref, wft_ref, bf_ref, out_ref):
    summed = acc_ref[:, 0:D]
    cnt = jnp.maximum(acc_ref[:, D:D + 1], 1.0)
    out_ref[...] = _mm(summed / cnt, wft_ref[...]) + bf_ref[...]


def _fin(acc, wft, bf):
    return pl.pallas_call(
        _fin_body,
        out_shape=jax.ShapeDtypeStruct((G, D), jnp.float32))(acc, wft, bf)


def kernel(edge_index, x1, x2, edge_feats, batch, params):
    p = params
    src = edge_index[0]
    dst = edge_index[1]

    def padcore(a, padval):
        halves = []
        for ci in range(NC):
            h = a[ci * EPC:(ci + 1) * EPC]
            pad = jnp.full((EPC_PAD - EPC,) + a.shape[1:], padval, a.dtype)
            halves.append(jnp.concatenate([h, pad], axis=0))
        return jnp.stack(halves)

    srcp = padcore(src, 0).reshape(NC, NS, CPW, CH)
    dstp = padcore(dst, N)
    dstp4 = dstp.reshape(NC, NS, CPW, CH)
    efp = padcore(edge_feats, 0.0)
    x1p = jnp.concatenate(
        [x1[:, 0], jnp.zeros((NPAD - N,), jnp.int32)])
    batch3 = batch.reshape(NB, 1, BLK)

    wnt = p["node"]["w"].T
    wgt = p["geom"]["w"].T
    bg = p["geom"]["b"].reshape(1, D)
    bn = p["node"]["b"].reshape(1, D)
    wet = p["edge"]["w"].T
    be = p["edge"]["b"].reshape(1, ED)

    lw = []
    for lp in p["layers"]:
        wm = lp["msg"]["w"]
        lw.append(dict(
            wit=wm[:, 0:D].T,
            wjt=wm[:, D:2 * D].T,
            we2t=wm[:, 2 * D:].T,
            bm=lp["msg"]["b"].reshape(1, D),
            wiht=lp["W_ih"].T,
            bih=lp["b_ih"].reshape(1, 3 * D),
            whht=lp["W_hh"].T,
            bhh=lp["b_hh"].reshape(1, 3 * D),
            gamma=lp["gamma"].reshape(1, D),
            beta=lp["beta"].reshape(1, D),
        ))

    t_table = _tmat(p["W_cat"])
    eatt_aug = _eatt(efp.reshape(NC * EPC_PAD, ED), wet, be)
    eagg_p, hcat = _sc_pre(
        eatt_aug.reshape(NC, EPC_PAD, D), dstp4, t_table, x1p)

    node, proj, ed = _enc(
        x2, hcat[:N], eagg_p[0], eagg_p[1],
        wgt, bg, wnt, bn, lw[0]["wjt"])

    for li, w in enumerate(lw):
        sp = _sc_spmv(proj, srcp, dstp4)
        h, stats = _post(node, sp[0], sp[1], ed, w["wit"], w["bm"],
                         w["we2t"], w["wiht"], w["bih"], w["whht"], w["bhh"])
        if li + 1 < len(lw):
            node, proj = _norm_proj(h, stats, w["gamma"], w["beta"],
                                    lw[li + 1]["wjt"])
        else:
            node = _norm_last(h, stats, w["gamma"], w["beta"])

    acc = _agg(node, batch3,
               p["agg_lin"]["w"].T, p["agg_lin"]["b"].reshape(1, D),
               p["agg_gate"]["w"].T, p["agg_gate"]["b"].reshape(1, D))
    graph = _fin(acc, p["agg_final"]["w"].T, p["agg_final"]["b"].reshape(1, D))
    return (node, graph)
